# baseline (speedup 1.0000x reference)
import functools
import jax
import jax.numpy as jnp
from jax import lax
from jax.experimental import pallas as pl
from jax.experimental.pallas import tpu as pltpu


def _embed_body(x_ref, w_ref, b_ref, tab_ref, adp_ref, out_ref, *,
                steps_per_day, n_tod, lb):
    w = w_ref[...]
    n_rows = tab_ref.shape[0]
    lanes = lax.broadcasted_iota(jnp.int32, (1, n_rows), 1)
    for li in range(lb):
        xv = x_ref[0, li]
        x1 = xv[:, 1:2]
        x2 = xv[:, 2:3]
        xp = (
            xv[:, 0:1] * w[0:1, :]
            + x1 * w[1:2, :]
            + x2 * w[2:3, :]
            + b_ref[...]
        )

        ti = (x1 * jnp.float32(steps_per_day)).astype(jnp.int32)
        di = x2.astype(jnp.int32) + n_tod
        oh = ((ti == lanes) | (di == lanes)).astype(jnp.float32)
        emb = jnp.dot(oh, tab_ref[...], preferred_element_type=jnp.float32)

        out_ref[0, li] = jnp.concatenate([xp, emb, adp_ref[li]], axis=-1)


def kernel(x, W_in, b_in, tod_table, dow_table, adp):
    B, L, N, D = x.shape
    E = W_in.shape[1]
    A = adp.shape[-1]
    OUT = E * 3 + A
    CHUNK = 2048
    LB = 4
    assert N % CHUNK == 0 and L % LB == 0

    b2 = b_in.reshape(1, E)
    n_tod = tod_table.shape[0]
    n_dow = dow_table.shape[0]
    tab = jnp.zeros((n_tod + n_dow, 2 * E), jnp.float32)
    tab = tab.at[:n_tod, :E].set(tod_table).at[n_tod:, E:].set(dow_table)

    grid = (L // LB, N // CHUNK, B)
    return pl.pallas_call(
        functools.partial(_embed_body, steps_per_day=288, n_tod=n_tod, lb=LB),
        grid=grid,
        in_specs=[
            pl.BlockSpec((1, LB, CHUNK, D), lambda l, c, b: (b, l, c, 0)),
            pl.BlockSpec((D, E), lambda l, c, b: (0, 0)),
            pl.BlockSpec((1, E), lambda l, c, b: (0, 0)),
            pl.BlockSpec(tab.shape, lambda l, c, b: (0, 0)),
            pl.BlockSpec((LB, CHUNK, A), lambda l, c, b: (l, c, 0)),
        ],
        out_specs=pl.BlockSpec((1, LB, CHUNK, OUT), lambda l, c, b: (b, l, c, 0)),
        out_shape=jax.ShapeDtypeStruct((B, L, N, OUT), jnp.float32),
    )(x, W_in, b2, tab, adp)


# speedup vs baseline: 1.0103x; 1.0103x over previous
import functools
import jax
import jax.numpy as jnp
from jax import lax
from jax.experimental import pallas as pl
from jax.experimental.pallas import tpu as pltpu


def _embed_body(x_ref, w_ref, b_ref, tab_ref, adp_ref, out_ref, *,
                steps_per_day, n_tod, lb):
    w = w_ref[...]
    n_rows = tab_ref.shape[0]
    lanes = lax.broadcasted_iota(jnp.int32, (1, n_rows), 1)
    for li in range(lb):
        xv = x_ref[0, li]
        x1 = xv[:, 1:2]
        x2 = xv[:, 2:3]
        xp = (
            xv[:, 0:1] * w[0:1, :]
            + x1 * w[1:2, :]
            + x2 * w[2:3, :]
            + b_ref[...]
        )

        ti = (x1 * jnp.float32(steps_per_day)).astype(jnp.int32)
        di = x2.astype(jnp.int32) + n_tod
        oh = ((ti == lanes) | (di == lanes)).astype(jnp.float32)
        emb = jnp.dot(oh, tab_ref[...], preferred_element_type=jnp.float32)

        out_ref[0, li] = jnp.concatenate([xp, emb, adp_ref[li]], axis=-1)


def kernel(x, W_in, b_in, tod_table, dow_table, adp):
    B, L, N, D = x.shape
    E = W_in.shape[1]
    A = adp.shape[-1]
    OUT = E * 3 + A
    CHUNK = 2048
    LB = 2
    assert N % CHUNK == 0 and L % LB == 0

    b2 = b_in.reshape(1, E)
    n_tod = tod_table.shape[0]
    n_dow = dow_table.shape[0]
    tab = jnp.zeros((n_tod + n_dow, 2 * E), jnp.float32)
    tab = tab.at[:n_tod, :E].set(tod_table).at[n_tod:, E:].set(dow_table)

    grid = (L // LB, N // CHUNK, B)
    return pl.pallas_call(
        functools.partial(_embed_body, steps_per_day=288, n_tod=n_tod, lb=LB),
        grid=grid,
        in_specs=[
            pl.BlockSpec((1, LB, CHUNK, D), lambda l, c, b: (b, l, c, 0)),
            pl.BlockSpec((D, E), lambda l, c, b: (0, 0)),
            pl.BlockSpec((1, E), lambda l, c, b: (0, 0)),
            pl.BlockSpec(tab.shape, lambda l, c, b: (0, 0)),
            pl.BlockSpec((LB, CHUNK, A), lambda l, c, b: (l, c, 0)),
        ],
        out_specs=pl.BlockSpec((1, LB, CHUNK, OUT), lambda l, c, b: (b, l, c, 0)),
        out_shape=jax.ShapeDtypeStruct((B, L, N, OUT), jnp.float32),
    )(x, W_in, b2, tab, adp)
